# baseline (speedup 1.0000x reference)
import functools

import jax
import jax.numpy as jnp
from jax import lax
from jax.experimental import pallas as pl
from jax.experimental.pallas import tpu as pltpu
from jax.experimental.pallas import tpu_sc as plsc

D_MODEL = 128
LA = 96
LB = 104


def _make_sc_kernel(B, L, D):
    info = plsc.get_sparse_core_info()
    NC, NS = info.num_cores, info.num_subcores
    NW = NC * NS
    rows_per_w = B // NW
    mesh = plsc.VectorSubcoreMesh(core_axis_name="c", subcore_axis_name="s")

    @functools.partial(
        pl.kernel,
        mesh=mesh,
        out_type=jax.ShapeDtypeStruct((B, L, D), jnp.float32),
        scratch_types=[
            pltpu.VMEM((LA,), jnp.int32),
            pltpu.VMEM((LB,), jnp.int32),
            pltpu.VMEM((LA, D), jnp.float32),
            pltpu.VMEM((LB, D), jnp.float32),
            pltpu.VMEM_SHARED((L, D), jnp.float32),
            pltpu.SemaphoreType.DMA,
            pltpu.SemaphoreType.DMA,
        ],
    )
    def k(x_hbm, pe_hbm, idx_hbm, out_hbm,
          idxa_v, idxb_v, buf0, buf1, pe_sh, sem0, sem1):
        sid = lax.axis_index("s")
        wid = sid * NC + lax.axis_index("c")

        @pl.when(sid == 0)
        def _():
            pltpu.sync_copy(pe_hbm, pe_sh)

        pltpu.sync_copy(idx_hbm.at[pl.ds(0, LA)], idxa_v)
        pltpu.sync_copy(idx_hbm.at[pl.ds(LA, LB)], idxb_v)
        plsc.subcore_barrier()

        def body(r, _):
            b = wid * rows_per_w + r
            pltpu.sync_copy(x_hbm.at[b, pl.ds(0, LA)], buf0)
            pltpu.sync_copy(x_hbm.at[b, pl.ds(LA, LB)], buf1)
            cp0 = pltpu.async_copy(pe_sh.at[idxa_v], buf0, sem0, add=True)
            cp1 = pltpu.async_copy(pe_sh.at[idxb_v], buf1, sem1, add=True)
            cp0.wait()
            cp1.wait()
            pltpu.sync_copy(buf0, out_hbm.at[b, pl.ds(0, LA)])
            pltpu.sync_copy(buf1, out_hbm.at[b, pl.ds(LA, LB)])
            return ()

        lax.fori_loop(0, rows_per_w, body, ())

    return k


def kernel(x, pe_table):
    B, L, D = x.shape
    pe = pe_table[:L]
    idx = jnp.arange(L, dtype=jnp.int32)
    return _make_sc_kernel(B, L, D)(x, pe, idx)


# speedup vs baseline: 1.6352x; 1.6352x over previous
import functools

import jax
import jax.numpy as jnp
from jax import lax
from jax.experimental import pallas as pl
from jax.experimental.pallas import tpu as pltpu
from jax.experimental.pallas import tpu_sc as plsc

D_MODEL = 128
LA = 96
LB = 104


def _make_sc_kernel(B, L, D):
    info = plsc.get_sparse_core_info()
    NC, NS = info.num_cores, info.num_subcores
    NW = NC * NS
    rows_per_w = B // NW
    mesh = plsc.VectorSubcoreMesh(core_axis_name="c", subcore_axis_name="s")

    NBUF = 3

    @functools.partial(
        pl.kernel,
        mesh=mesh,
        out_type=jax.ShapeDtypeStruct((B, L, D), jnp.float32),
        scratch_types=[
            pltpu.VMEM((LA,), jnp.int32),
            pltpu.VMEM((LB,), jnp.int32),
            pltpu.VMEM((NBUF, LA, D), jnp.float32),
            pltpu.VMEM((NBUF, LB, D), jnp.float32),
            pltpu.VMEM_SHARED((L, D), jnp.float32),
        ]
        + [pltpu.SemaphoreType.DMA] * (3 * NBUF),
    )
    def k(x_hbm, pe_hbm, idx_hbm, out_hbm,
          idxa_v, idxb_v, bufa, bufb, pe_sh, *sems):
        sem_in = sems[0:NBUF]
        sem_add = sems[NBUF:2 * NBUF]
        sem_out = sems[2 * NBUF:3 * NBUF]
        sid = lax.axis_index("s")
        wid = sid * NC + lax.axis_index("c")
        base = wid * rows_per_w

        @pl.when(sid == 0)
        def _():
            pltpu.sync_copy(pe_hbm, pe_sh)

        pltpu.sync_copy(idx_hbm.at[pl.ds(0, LA)], idxa_v)
        pltpu.sync_copy(idx_hbm.at[pl.ds(LA, LB)], idxb_v)
        plsc.subcore_barrier()

        def start_in(r, p):
            b = base + r
            return (
                pltpu.async_copy(x_hbm.at[b, pl.ds(0, LA)], bufa.at[p], sem_in[p]),
                pltpu.async_copy(x_hbm.at[b, pl.ds(LA, LB)], bufb.at[p], sem_in[p]),
            )

        def start_add(p):
            return (
                pltpu.async_copy(pe_sh.at[idxa_v], bufa.at[p], sem_add[p], add=True),
                pltpu.async_copy(pe_sh.at[idxb_v], bufb.at[p], sem_add[p], add=True),
            )

        def start_out(r, p):
            b = base + r
            return (
                pltpu.async_copy(bufa.at[p], out_hbm.at[b, pl.ds(0, LA)], sem_out[p]),
                pltpu.async_copy(bufb.at[p], out_hbm.at[b, pl.ds(LA, LB)], sem_out[p]),
            )

        h_in = [None] * NBUF
        h_add = [None] * NBUF
        h_out = [None] * NBUF
        for step in range(rows_per_w + 2):
            r_in, r_add, r_out = step, step - 1, step - 2
            if r_in < rows_per_w:
                p = r_in % NBUF
                if h_out[p] is not None:
                    for h in h_out[p]:
                        h.wait()
                h_in[p] = start_in(r_in, p)
            if 0 <= r_add < rows_per_w:
                p = r_add % NBUF
                for h in h_in[p]:
                    h.wait()
                h_add[p] = start_add(p)
            if 0 <= r_out:
                p = r_out % NBUF
                for h in h_add[p]:
                    h.wait()
                h_out[p] = start_out(r_out, p)
        for r_out in (rows_per_w - 3, rows_per_w - 2, rows_per_w - 1):
            p = r_out % NBUF
            for h in h_out[p]:
                h.wait()

    return k


def kernel(x, pe_table):
    B, L, D = x.shape
    pe = pe_table[:L]
    idx = jnp.arange(L, dtype=jnp.int32)
    return _make_sc_kernel(B, L, D)(x, pe, idx)
